# baseline (device time: 81269 ns/iter reference)
import functools

import jax
import jax.numpy as jnp
from jax import lax
from jax.experimental import pallas as pl
from jax.experimental.pallas import tpu as pltpu

N_DEV = 4
S = 4


def kernel(partial, gamma):
    m_total, d = partial.shape[-2], partial.shape[-1]
    m_per = m_total // N_DEV
    m_half = m_per // 2
    sub_m = m_half // S

    def body(x_hbm, g_hbm, out_ref, comm_a, comm_b, xa, xb, gv,
             send_a, recv_a, send_b, recv_b, load_sems, ballast):
        my = lax.axis_index("i")
        left = lax.rem(my + N_DEV - 1, N_DEV)
        right = lax.rem(my + 1, N_DEV)

        def a_chunk(h):
            return lax.rem(my + 2 * N_DEV - 2 - h, N_DEV)

        def b_chunk(h):
            return lax.rem(my + 2 + h, N_DEV)

        def half_rows(r, c):
            return pl.ds(c * m_per + r * m_half, m_half)

        def slot(h, s):
            return h * S + s

        rings = (
            (comm_a, xa, send_a, recv_a, right, a_chunk, 0),
            (comm_b, xb, send_b, recv_b, left, b_chunk, m_half),
        )

        def mk(comm, snd, rcv, dst_dev, h, s, src_ref):
            return pltpu.make_async_remote_copy(
                src_ref=src_ref,
                dst_ref=comm.at[slot(h, s)],
                send_sem=snd.at[h * S + s],
                recv_sem=rcv.at[h * S + s],
                device_id=(dst_dev,),
                device_id_type=pl.DeviceIdType.MESH,
            )

        gload = pltpu.make_async_copy(g_hbm, gv, load_sems.at[2 * (N_DEV - 1)])
        gload.start()
        loads = {}
        for r, (_, xr, _, _, _, chunk, _) in enumerate(rings):
            for h in range(N_DEV - 1):
                ld = pltpu.make_async_copy(
                    x_hbm.at[0, half_rows(r, chunk(h)), :],
                    xr.at[h],
                    load_sems.at[r * (N_DEV - 1) + h],
                )
                ld.start()
                loads[(r, h)] = ld

        barrier_sem = pltpu.get_barrier_semaphore()
        for nbr in [left, right]:
            pl.semaphore_signal(
                barrier_sem, inc=1,
                device_id=(nbr,), device_id_type=pl.DeviceIdType.MESH,
            )
        pl.semaphore_wait(barrier_sem, 2)
        gload.wait()

        descs = {}
        ca0 = lax.rem(my + N_DEV - 1, N_DEV)
        cb0 = lax.rem(my + 1, N_DEV)
        for s in range(S):
            for r, (comm, _, snd, rcv, dev, _, _) in enumerate(rings):
                c0 = ca0 if r == 0 else cb0
                src = x_hbm.at[0, pl.ds((c0 * m_per + r * m_half) + s * sub_m,
                                        sub_m), :]
                rd = mk(comm, snd, rcv, dev, 0, s, src)
                rd.start()
                descs[(r, 0, s)] = rd

        for h in range(N_DEV - 1):
            for s in range(S):
                for r, (comm, xr, snd, rcv, dev, chunk, row0) in enumerate(
                    rings
                ):
                    descs[(r, h, s)].wait_recv()
                    if (r, h) in loads:
                        loads.pop((r, h)).wait()
                    lvl = slot(h, s)
                    xs = xr[h, pl.ds(s * sub_m, sub_m), :]
                    if h < N_DEV - 2:
                        comm[lvl] += xs
                        nd = mk(comm, snd, rcv, dev, h + 1, s, comm.at[lvl])
                        nd.start()
                        descs[(r, h + 1, s)] = nd
                    else:
                        y = comm[lvl] + xs
                        inv = lax.rsqrt(
                            jnp.mean(y * y, axis=-1, keepdims=True) + 1e-6
                        )
                        out_ref[pl.ds(row0 + s * sub_m, sub_m), :] = (
                            y * inv * gv[:, :]
                        )

        for rd in descs.values():
            rd.wait_send()

        @functools.partial(pl.run_scoped, sem2=pltpu.SemaphoreType.REGULAR)
        def _(sem2):
            for nbr in [left, right]:
                pl.semaphore_signal(
                    sem2, inc=1,
                    device_id=(nbr,), device_id_type=pl.DeviceIdType.MESH,
                )
            pl.semaphore_wait(sem2, 2)

    n_slots = (N_DEV - 1) * S
    n_sems = (N_DEV - 1) * S
    return pl.pallas_call(
        body,
        out_shape=jax.ShapeDtypeStruct((m_per, d), jnp.float32),
        in_specs=[
            pl.BlockSpec(memory_space=pltpu.HBM),
            pl.BlockSpec(memory_space=pltpu.HBM),
        ],
        out_specs=pl.BlockSpec(memory_space=pltpu.VMEM),
        scratch_shapes=[
            pltpu.VMEM((n_slots, sub_m, d), jnp.float32),
            pltpu.VMEM((n_slots, sub_m, d), jnp.float32),
            pltpu.VMEM((N_DEV - 1, m_half, d), jnp.float32),
            pltpu.VMEM((N_DEV - 1, m_half, d), jnp.float32),
            pltpu.VMEM((1, d), jnp.float32),
            pltpu.SemaphoreType.DMA((n_sems,)),
            pltpu.SemaphoreType.DMA((n_sems,)),
            pltpu.SemaphoreType.DMA((n_sems,)),
            pltpu.SemaphoreType.DMA((n_sems,)),
            pltpu.SemaphoreType.DMA((2 * (N_DEV - 1) + 1,)),
            pltpu.VMEM((6 * 1024, 1024), jnp.float32),
        ],
        compiler_params=pltpu.CompilerParams(
            collective_id=0, vmem_limit_bytes=100 * 1024 * 1024,
        ),
    )(partial, gamma.reshape(1, -1))


# device time: 79521 ns/iter; 1.0220x vs baseline; 1.0220x over previous
import jax
import jax.numpy as jnp
from jax import lax
from jax.experimental import pallas as pl
from jax.experimental.pallas import tpu as pltpu

N_DEV = 4
S = 4


def kernel(partial, gamma):
    m_total, d = partial.shape[-2], partial.shape[-1]
    m_per = m_total // N_DEV
    m_half = m_per // 2
    sub_m = m_half // S

    def body(x_hbm, g_hbm, out_hbm, comm_a, comm_b, xa, xb, gv, outv,
             send_a, recv_a, send_b, recv_b, load_sems, store_sems,
             ballast):
        my = lax.axis_index("i")
        left = lax.rem(my + N_DEV - 1, N_DEV)
        right = lax.rem(my + 1, N_DEV)

        def a_chunk(h):
            return lax.rem(my + 2 * N_DEV - 2 - h, N_DEV)

        def b_chunk(h):
            return lax.rem(my + 2 + h, N_DEV)

        def half_rows(r, c):
            return pl.ds(c * m_per + r * m_half, m_half)

        def slot(h, s):
            return h * S + s

        rings = (
            (comm_a, xa, send_a, recv_a, right, a_chunk, 0),
            (comm_b, xb, send_b, recv_b, left, b_chunk, m_half),
        )

        def mk(comm, snd, rcv, dst_dev, h, s, src_ref):
            return pltpu.make_async_remote_copy(
                src_ref=src_ref,
                dst_ref=comm.at[slot(h, s)],
                send_sem=snd.at[h * S + s],
                recv_sem=rcv.at[h * S + s],
                device_id=(dst_dev,),
                device_id_type=pl.DeviceIdType.MESH,
            )

        gload = pltpu.make_async_copy(g_hbm, gv, load_sems.at[2 * (N_DEV - 1)])
        gload.start()
        loads = {}
        for r, (_, xr, _, _, _, chunk, _) in enumerate(rings):
            for h in range(N_DEV - 1):
                ld = pltpu.make_async_copy(
                    x_hbm.at[0, half_rows(r, chunk(h)), :],
                    xr.at[h],
                    load_sems.at[r * (N_DEV - 1) + h],
                )
                ld.start()
                loads[(r, h)] = ld

        barrier_sem = pltpu.get_barrier_semaphore()
        for nbr in [left, right]:
            pl.semaphore_signal(
                barrier_sem, inc=1,
                device_id=(nbr,), device_id_type=pl.DeviceIdType.MESH,
            )
        pl.semaphore_wait(barrier_sem, 2)
        gload.wait()

        descs = {}
        ca0 = lax.rem(my + N_DEV - 1, N_DEV)
        cb0 = lax.rem(my + 1, N_DEV)
        for s in range(S):
            for r, (comm, _, snd, rcv, dev, _, _) in enumerate(rings):
                c0 = ca0 if r == 0 else cb0
                src = x_hbm.at[0, pl.ds((c0 * m_per + r * m_half) + s * sub_m,
                                        sub_m), :]
                rd = mk(comm, snd, rcv, dev, 0, s, src)
                rd.start()
                descs[(r, 0, s)] = rd

        stores = []
        for h in range(N_DEV - 1):
            for s in range(S):
                for r, (comm, xr, snd, rcv, dev, chunk, row0) in enumerate(
                    rings
                ):
                    descs[(r, h, s)].wait_recv()
                    if (r, h) in loads:
                        loads.pop((r, h)).wait()
                    lvl = slot(h, s)
                    xs = xr[h, pl.ds(s * sub_m, sub_m), :]
                    if h < N_DEV - 2:
                        comm[lvl] += xs
                        nd = mk(comm, snd, rcv, dev, h + 1, s, comm.at[lvl])
                        nd.start()
                        descs[(r, h + 1, s)] = nd
                    else:
                        y = comm[lvl] + xs
                        inv = lax.rsqrt(
                            jnp.mean(y * y, axis=-1, keepdims=True) + 1e-6
                        )
                        rows = pl.ds(row0 + s * sub_m, sub_m)
                        outv[rows, :] = y * inv * gv[:, :]
                        st = pltpu.make_async_copy(
                            outv.at[rows, :],
                            out_hbm.at[rows, :],
                            store_sems.at[r * S + s],
                        )
                        st.start()
                        stores.append(st)

        for st in stores:
            st.wait()
        for rd in descs.values():
            rd.wait_send()


    n_slots = (N_DEV - 1) * S
    n_sems = (N_DEV - 1) * S
    return pl.pallas_call(
        body,
        out_shape=jax.ShapeDtypeStruct((m_per, d), jnp.float32),
        in_specs=[
            pl.BlockSpec(memory_space=pltpu.HBM),
            pl.BlockSpec(memory_space=pltpu.HBM),
        ],
        out_specs=pl.BlockSpec(memory_space=pltpu.HBM),
        scratch_shapes=[
            pltpu.VMEM((n_slots, sub_m, d), jnp.float32),
            pltpu.VMEM((n_slots, sub_m, d), jnp.float32),
            pltpu.VMEM((N_DEV - 1, m_half, d), jnp.float32),
            pltpu.VMEM((N_DEV - 1, m_half, d), jnp.float32),
            pltpu.VMEM((1, d), jnp.float32),
            pltpu.VMEM((m_per, d), jnp.float32),
            pltpu.SemaphoreType.DMA((n_sems,)),
            pltpu.SemaphoreType.DMA((n_sems,)),
            pltpu.SemaphoreType.DMA((n_sems,)),
            pltpu.SemaphoreType.DMA((n_sems,)),
            pltpu.SemaphoreType.DMA((2 * (N_DEV - 1) + 1,)),
            pltpu.SemaphoreType.DMA((2 * S,)),
            pltpu.VMEM((6 * 1024, 1024), jnp.float32),
        ],
        compiler_params=pltpu.CompilerParams(
            collective_id=0, vmem_limit_bytes=100 * 1024 * 1024,
        ),
    )(partial, gamma.reshape(1, -1))
